# baseline (speedup 1.0000x reference)
import jax
import jax.numpy as jnp
from jax.experimental import pallas as pl
from jax.experimental.pallas import tpu as pltpu

_ALPHA = 0.25
_EPS = 1e-9

_B, _C, _H, _W = 16, 1, 512, 512
_BH = 256
_GRID = _B * (_H // _BH)
_N = float(_B * _C * _H * _W)


def _body(coeff_ref, cls_ref, reg_ref, tgt_ref, out_ref, fa, sa, ta, ca):
    i = pl.program_id(0)

    @pl.when(i == 0)
    def _():
        z = jnp.zeros((8, _W), jnp.float32)
        fa[...] = z
        sa[...] = z
        ta[...] = z
        ca[...] = z

    for j in range(_BH // 8):
        rows = pl.ds(j * 8, 8)
        c = cls_ref[0, 0, rows, :]
        r = reg_ref[0, 0, rows, :]
        t = tgt_ref[0, 0, rows, :]
        pos = t > 0.0
        one_m = 1.0 - c
        l1 = jnp.log(c + _EPS)
        l2 = jnp.log(one_m + _EPS)
        focal = jnp.where(pos, (-_ALPHA) * (one_m * one_m) * l1,
                          (_ALPHA - 1.0) * (c * c) * l2)
        d = r - t
        sq = d * d
        fa[...] += focal
        sa[...] += jnp.where(pos, sq, 0.0)
        ta[...] += sq
        ca[...] += jnp.where(pos, 1.0, 0.0)

    @pl.when(i == _GRID - 1)
    def _():
        foc = jnp.sum(fa[...])
        fsq = jnp.sum(sa[...])
        tsq = jnp.sum(ta[...])
        cnt = jnp.sum(ca[...])
        fc = jnp.maximum(cnt, 1.0)
        uc = jnp.maximum(_N - cnt, 1.0)
        loss_cls = foc / _N
        lrf = fsq / fc
        lru = (tsq - fsq) / uc
        lr = 20.0 * lrf + lru
        out_ref[0] = lr + coeff_ref[0] * loss_cls
        out_ref[1] = lr
        out_ref[2] = lrf
        out_ref[3] = lru
        out_ref[4] = loss_cls


def kernel(cls, reg, targets, epoch):
    coeff = jnp.where(jnp.asarray(epoch) < 500, 10.0, 0.1).astype(
        jnp.float32).reshape(1)

    nh = _H // _BH
    blk = (1, 1, _BH, _W)
    spec = pl.BlockSpec(blk, lambda i: (i // nh, 0, i % nh, 0))
    out = pl.pallas_call(
        _body,
        grid=(_GRID,),
        in_specs=[pl.BlockSpec(memory_space=pltpu.SMEM), spec, spec, spec],
        out_specs=pl.BlockSpec(memory_space=pltpu.SMEM),
        out_shape=jax.ShapeDtypeStruct((5,), jnp.float32),
        scratch_shapes=[
            pltpu.VMEM((8, _W), jnp.float32),
            pltpu.VMEM((8, _W), jnp.float32),
            pltpu.VMEM((8, _W), jnp.float32),
            pltpu.VMEM((8, _W), jnp.float32),
        ],
        compiler_params=pltpu.CompilerParams(
            dimension_semantics=("arbitrary",)),
    )(coeff, cls, reg, targets)

    return (out[0], out[1], out[2], out[3], out[4])


# speedup vs baseline: 1.4720x; 1.4720x over previous
import jax
import jax.numpy as jnp
from jax.experimental import pallas as pl
from jax.experimental.pallas import tpu as pltpu

_ALPHA = 0.25
_EPS = 1e-9

_B, _C, _H, _W = 16, 1, 512, 512
_BB = 2
_GRID = _B // _BB
_N = float(_B * _C * _H * _W)


def _body(coeff_ref, cls_ref, reg_ref, tgt_ref, out_ref, fa, sa, ta, ca):
    i = pl.program_id(0)

    @pl.when(i == 0)
    def _():
        z = jnp.zeros((8, _W), jnp.float32)
        fa[...] = z
        sa[...] = z
        ta[...] = z
        ca[...] = z

    for j in range(_BB * _H // 8):
        rows = pl.ds(j * 8, 8)
        c = cls_ref[rows, :]
        r = reg_ref[rows, :]
        t = tgt_ref[rows, :]
        pos = t > 0.0
        one_m = 1.0 - c
        l1 = jnp.log(c + _EPS)
        l2 = jnp.log(one_m + _EPS)
        focal = jnp.where(pos, (-_ALPHA) * (one_m * one_m) * l1,
                          (_ALPHA - 1.0) * (c * c) * l2)
        d = r - t
        sq = d * d
        fa[...] += focal
        sa[...] += jnp.where(pos, sq, 0.0)
        ta[...] += sq
        ca[...] += jnp.where(pos, 1.0, 0.0)

    @pl.when(i == _GRID - 1)
    def _():
        foc = jnp.sum(fa[...])
        fsq = jnp.sum(sa[...])
        tsq = jnp.sum(ta[...])
        cnt = jnp.sum(ca[...])
        fc = jnp.maximum(cnt, 1.0)
        uc = jnp.maximum(_N - cnt, 1.0)
        loss_cls = foc / _N
        lrf = fsq / fc
        lru = (tsq - fsq) / uc
        lr = 20.0 * lrf + lru
        out_ref[0] = lr + coeff_ref[0] * loss_cls
        out_ref[1] = lr
        out_ref[2] = lrf
        out_ref[3] = lru
        out_ref[4] = loss_cls


def kernel(cls, reg, targets, epoch):
    coeff = jnp.where(jnp.asarray(epoch) < 500, 10.0, 0.1).astype(
        jnp.float32).reshape(1)

    blk = (_BB * _H, _W)
    spec = pl.BlockSpec(blk, lambda i: (i, 0))
    out = pl.pallas_call(
        _body,
        grid=(_GRID,),
        in_specs=[pl.BlockSpec(memory_space=pltpu.SMEM), spec, spec, spec],
        out_specs=pl.BlockSpec(memory_space=pltpu.SMEM),
        out_shape=jax.ShapeDtypeStruct((5,), jnp.float32),
        scratch_shapes=[
            pltpu.VMEM((8, _W), jnp.float32),
            pltpu.VMEM((8, _W), jnp.float32),
            pltpu.VMEM((8, _W), jnp.float32),
            pltpu.VMEM((8, _W), jnp.float32),
        ],
        compiler_params=pltpu.CompilerParams(
            dimension_semantics=("arbitrary",)),
    )(coeff, cls.reshape(_B * _H, _W), reg.reshape(_B * _H, _W),
      targets.reshape(_B * _H, _W))

    return (out[0], out[1], out[2], out[3], out[4])


# speedup vs baseline: 1.5220x; 1.0339x over previous
import jax
import jax.numpy as jnp
from jax.experimental import pallas as pl
from jax.experimental.pallas import tpu as pltpu

_ALPHA = 0.25
_EPS = 1e-9

_B, _C, _H, _W = 16, 1, 512, 512
_BB = 4
_GRID = _B // _BB
_N = float(_B * _C * _H * _W)


def _body(coeff_ref, cls_ref, reg_ref, tgt_ref, out_ref, fa, sa, ta, ca):
    i = pl.program_id(0)

    @pl.when(i == 0)
    def _():
        z = jnp.zeros((8, _W), jnp.float32)
        fa[...] = z
        sa[...] = z
        ta[...] = z
        ca[...] = z

    for j in range(_BB * _H // 8):
        rows = pl.ds(j * 8, 8)
        c = cls_ref[rows, :]
        r = reg_ref[rows, :]
        t = tgt_ref[rows, :]
        pos = t > 0.0
        one_m = 1.0 - c
        l1 = jnp.log(c + _EPS)
        l2 = jnp.log(one_m + _EPS)
        focal = jnp.where(pos, (-_ALPHA) * (one_m * one_m) * l1,
                          (_ALPHA - 1.0) * (c * c) * l2)
        d = r - t
        sq = d * d
        fa[...] += focal
        sa[...] += jnp.where(pos, sq, 0.0)
        ta[...] += sq
        ca[...] += jnp.where(pos, 1.0, 0.0)

    @pl.when(i == _GRID - 1)
    def _():
        foc = jnp.sum(fa[...])
        fsq = jnp.sum(sa[...])
        tsq = jnp.sum(ta[...])
        cnt = jnp.sum(ca[...])
        fc = jnp.maximum(cnt, 1.0)
        uc = jnp.maximum(_N - cnt, 1.0)
        loss_cls = foc / _N
        lrf = fsq / fc
        lru = (tsq - fsq) / uc
        lr = 20.0 * lrf + lru
        out_ref[0] = lr + coeff_ref[0] * loss_cls
        out_ref[1] = lr
        out_ref[2] = lrf
        out_ref[3] = lru
        out_ref[4] = loss_cls


def kernel(cls, reg, targets, epoch):
    coeff = jnp.where(jnp.asarray(epoch) < 500, 10.0, 0.1).astype(
        jnp.float32).reshape(1)

    blk = (_BB * _H, _W)
    spec = pl.BlockSpec(blk, lambda i: (i, 0))
    out = pl.pallas_call(
        _body,
        grid=(_GRID,),
        in_specs=[pl.BlockSpec(memory_space=pltpu.SMEM), spec, spec, spec],
        out_specs=pl.BlockSpec(memory_space=pltpu.SMEM),
        out_shape=jax.ShapeDtypeStruct((5,), jnp.float32),
        scratch_shapes=[
            pltpu.VMEM((8, _W), jnp.float32),
            pltpu.VMEM((8, _W), jnp.float32),
            pltpu.VMEM((8, _W), jnp.float32),
            pltpu.VMEM((8, _W), jnp.float32),
        ],
        compiler_params=pltpu.CompilerParams(
            dimension_semantics=("arbitrary",)),
    )(coeff, cls.reshape(_B * _H, _W), reg.reshape(_B * _H, _W),
      targets.reshape(_B * _H, _W))

    return (out[0], out[1], out[2], out[3], out[4])
